# baseline (speedup 1.0000x reference)
import functools

import jax
import jax.numpy as jnp
from jax import lax
from jax.experimental import pallas as pl
from jax.experimental.pallas import tpu as pltpu
from jax.experimental.pallas import tpu_sc as plsc


def _gather_call(B, D, NC, NW, b_per_w):
    mesh = plsc.VectorSubcoreMesh(core_axis_name="c", subcore_axis_name="s")

    @functools.partial(
        pl.kernel,
        mesh=mesh,
        out_type=jax.ShapeDtypeStruct((B, D), jnp.float32),
        scratch_types=[
            pltpu.VMEM((b_per_w,), jnp.int32),
            pltpu.VMEM((b_per_w, D), jnp.float32),
            pltpu.SemaphoreType.DMA,
        ],
        compiler_params=pltpu.CompilerParams(use_tc_tiling_on_sc=False),
    )
    def k(table_hbm, idx_hbm, out_hbm, idx_v, rows_v, sem):
        wid = lax.axis_index("s") * NC + lax.axis_index("c")
        base = wid * b_per_w
        pltpu.sync_copy(idx_hbm.at[pl.ds(base, b_per_w)], idx_v)
        pltpu.async_copy(table_hbm.at[idx_v], rows_v, sem).wait()
        pltpu.sync_copy(rows_v, out_hbm.at[pl.ds(base, b_per_w)])

    return k


def kernel(raw_X, table):
    B = raw_X.shape[0]
    V, D = table.shape
    info = plsc.get_sparse_core_info()
    NC, NS = info.num_cores, info.num_subcores
    NW = NC * NS
    b_per_w = B // NW
    idx = raw_X[:, 1].astype(jnp.int32)
    return _gather_call(B, D, NC, NW, b_per_w)(table, idx)


# speedup vs baseline: 2.6674x; 2.6674x over previous
import functools

import jax
import jax.numpy as jnp
from jax import lax
from jax.experimental import pallas as pl
from jax.experimental.pallas import tpu as pltpu
from jax.experimental.pallas import tpu_sc as plsc

L = 16
CHUNK = 16384


def _pop(mask):
    return plsc.all_reduce_population_count(mask)[0]


def _emb_call(B, V, D, NC, NS):
    mesh = plsc.VectorSubcoreMesh(core_axis_name="c", subcore_axis_name="s")
    half = D // NC
    bpt = B // NS
    aligned_v = (V // 128) * 128
    n_full = aligned_v // CHUNK
    rows2 = aligned_v - n_full * CHUNK
    lo2 = n_full * CHUNK
    t_rows = V - aligned_v
    cwords = half * CHUNK
    tail_off = 2 * cwords

    @functools.partial(
        pl.kernel,
        mesh=mesh,
        out_type=jax.ShapeDtypeStruct((D, B), jnp.float32),
        scratch_types=[
            pltpu.VMEM((bpt,), jnp.int32),
            pltpu.VMEM((bpt + L,), jnp.int32),
            pltpu.VMEM((L * half,), jnp.int32),
            pltpu.VMEM((L * half,), jnp.float32),
            pltpu.VMEM((half, bpt), jnp.float32),
            pltpu.VMEM_SHARED((2 * cwords + 128 * 128,), jnp.float32),
            pltpu.SemaphoreType.DMA,
            pltpu.SemaphoreType.DMA,
        ],
        compiler_params=pltpu.CompilerParams(needs_layout_passes=False),
    )
    def k(t3_hbm, tail_hbm, idx_hbm, outT_hbm, idx_v, run_v, widx_v, wbuf_v,
          colbuf, chunk_s, sem_st, sem_g):
        cid = lax.axis_index("c")
        sid = lax.axis_index("s")
        base = sid * bpt
        pltpu.sync_copy(idx_hbm.at[pl.ds(base, bpt)], idx_v)

        iota = lax.iota(jnp.int32, L)

        def stage(lo, buf_off, rows):
            for j in range(2):
                c_loc = sid * 2 + j
                a = cid * (half // 8) + c_loc // 8
                c2 = c_loc % 8
                pltpu.async_copy(
                    t3_hbm.at[a, c2, pl.ds(lo, rows)],
                    chunk_s.at[pl.ds(buf_off + c_loc * rows, rows)],
                    sem_st,
                )

        def wait_stage(rows, count=2):
            for _ in range(count):
                pltpu.make_async_copy(
                    t3_hbm.at[0, 0, pl.ds(0, rows)],
                    chunk_s.at[pl.ds(0, rows)],
                    sem_st,
                ).wait()

        def process(buf_off, lo, rows, sr, sc):
            def scan_body(g, off):
                v = idx_v[pl.ds(g * L, L)]
                m = (v >= lo) & (v < lo + rows)
                packed = (v - lo) + (iota + g * L) * CHUNK
                plsc.store_compressed(run_v.at[pl.ds(off, L)], packed, mask=m)
                return off + _pop(m)

            n = lax.fori_loop(0, bpt // L, scan_body, 0)

            def batch_cond(j):
                return j < n

            def batch_body(j):
                pv = run_v[pl.ds(j, L)]
                rloc = pv & (CHUNK - 1)
                dpos = pv >> 14
                valid = (iota + j) < n
                rloc = jnp.where(valid, rloc, 0)
                for c in range(half):
                    widx_v[pl.ds(c * L, L)] = rloc * sr + (buf_off + c * sc)
                for q in range(L * half // 128):
                    pltpu.async_copy(
                        chunk_s.at[widx_v.at[pl.ds(q * 128, 128)]],
                        wbuf_v.at[pl.ds(q * 128, 128)],
                        sem_g,
                    )
                for q in range(L * half // 128):
                    pltpu.make_async_copy(
                        t3_hbm.at[0, 0, pl.ds(0, 128)],
                        wbuf_v.at[pl.ds(0, 128)],
                        sem_g,
                    ).wait()
                for c in range(half):
                    vals = wbuf_v[pl.ds(c * L, L)]
                    ci = iota * 0 + c
                    plsc.store_scatter(colbuf, [ci, dpos], vals, mask=valid)
                return j + L

            lax.while_loop(batch_cond, batch_body, 0)

        stage(0, 0, CHUNK)

        @pl.when(sid == 0)
        def _():
            for r in range(128):
                pltpu.async_copy(
                    tail_hbm.at[r, :],
                    chunk_s.at[pl.ds(tail_off + r * 128, 128)],
                    sem_st,
                )
            wait_stage(128, count=128)

        wait_stage(CHUNK)
        plsc.subcore_barrier()

        def epoch(ep, carry):
            parity = ep & 1
            nxt_off = (1 - parity) * cwords

            @pl.when(ep < n_full - 1)
            def _():
                stage((ep + 1) * CHUNK, nxt_off, CHUNK)

            @pl.when(ep == n_full - 1)
            def _():
                stage(lo2, nxt_off, rows2)

            process(parity * cwords, ep * CHUNK, CHUNK, 1, CHUNK)

            @pl.when(ep < n_full - 1)
            def _():
                wait_stage(CHUNK)

            @pl.when(ep == n_full - 1)
            def _():
                wait_stage(rows2)

            plsc.subcore_barrier()
            return carry

        lax.fori_loop(0, n_full, epoch, 0)

        process((n_full & 1) * cwords, lo2, rows2, 1, rows2)
        process(tail_off + cid * half, aligned_v, t_rows, 128, 1)

        pltpu.sync_copy(
            colbuf,
            outT_hbm.at[pl.ds(cid * half, half), pl.ds(base, bpt)],
        )

    return k


def kernel(raw_X, table):
    B = raw_X.shape[0]
    V, D = table.shape
    info = plsc.get_sparse_core_info()
    NC, NS = info.num_cores, info.num_subcores
    idx = raw_X[:, 1].astype(jnp.int32)
    t3 = table.T.reshape(D // 8, 8, V)
    aligned_v = (V // 128) * 128
    tail = jnp.pad(
        table[aligned_v:], ((0, 128 - (V - aligned_v)), (0, 128 - D))
    )
    outT = _emb_call(B, V, D, NC, NS)(t3, tail, idx)
    return outT.T


# speedup vs baseline: 3.1526x; 1.1819x over previous
import functools

import jax
import jax.numpy as jnp
from jax import lax
from jax.experimental import pallas as pl
from jax.experimental.pallas import tpu as pltpu
from jax.experimental.pallas import tpu_sc as plsc

L = 16
CHUNK = 11648
PKB = 14
PK = 1 << PKB
NBUF = 4


def _pop(mask):
    return plsc.all_reduce_population_count(mask)[0]


def _emb_call(B, V, D, NC, NS):
    mesh = plsc.VectorSubcoreMesh(core_axis_name="c", subcore_axis_name="s")
    half = D // NC
    bpt = B // NS
    aligned_v = (V // 128) * 128
    n_full = aligned_v // CHUNK
    rows2 = aligned_v - n_full * CHUNK
    n_all = n_full + 1
    n_loop = (n_all - 2) // NBUF * NBUF
    t_rows = V - aligned_v
    cwords = half * CHUNK
    tail_off = NBUF * cwords

    def rows_of(e):
        return CHUNK if e < n_full else rows2

    @functools.partial(
        pl.kernel,
        mesh=mesh,
        out_type=jax.ShapeDtypeStruct((D, B), jnp.float32),
        scratch_types=[
            pltpu.VMEM((bpt,), jnp.int32),
            pltpu.VMEM((bpt + L,), jnp.int32),
            pltpu.VMEM((L * half,), jnp.int32),
            pltpu.VMEM((L * half,), jnp.float32),
            pltpu.VMEM((half, bpt), jnp.float32),
            pltpu.VMEM_SHARED((NBUF * cwords + 128 * 128,), jnp.float32),
            pltpu.SemaphoreType.DMA,
            pltpu.SemaphoreType.DMA,
            pltpu.SemaphoreType.DMA,
            pltpu.SemaphoreType.DMA,
            pltpu.SemaphoreType.DMA,
        ],
        compiler_params=pltpu.CompilerParams(needs_layout_passes=False),
    )
    def k(t3_hbm, tail_hbm, idx_hbm, outT_hbm, idx_v, run_v, widx_v, wbuf_v,
          colbuf, chunk_s, sem0, sem1, sem2, sem3, sem_g):
        sems = [sem0, sem1, sem2, sem3]
        cid = lax.axis_index("c")
        sid = lax.axis_index("s")
        base = sid * bpt
        pltpu.sync_copy(idx_hbm.at[pl.ds(base, bpt)], idx_v)

        iota = lax.iota(jnp.int32, L)

        def stage(lo, slot, rows, sem):
            for j in range(2):
                c_loc = sid * 2 + j
                a = cid * (half // 8) + c_loc // 8
                c2 = c_loc % 8
                pltpu.async_copy(
                    t3_hbm.at[a, c2, pl.ds(lo, rows)],
                    chunk_s.at[pl.ds(slot * cwords + c_loc * rows, rows)],
                    sem,
                )

        def wait_stage(rows, sem, count=2):
            for _ in range(count):
                pltpu.make_async_copy(
                    t3_hbm.at[0, 0, pl.ds(0, rows)],
                    chunk_s.at[pl.ds(0, rows)],
                    sem,
                ).wait()

        def process(buf_off, lo, rows, sr, sc):
            def scan_body(g, off):
                v = idx_v[pl.ds(g * L, L)]
                m = (v >= lo) & (v < lo + rows)
                packed = (v - lo) + (iota + g * L) * PK
                plsc.store_compressed(run_v.at[pl.ds(off, L)], packed, mask=m)
                return off + _pop(m)

            n = lax.fori_loop(0, bpt // L, scan_body, 0)

            def batch_cond(j):
                return j < n

            def batch_body(j):
                pv = run_v[pl.ds(j, L)]
                rloc = pv & (PK - 1)
                dpos = pv >> PKB
                valid = (iota + j) < n
                rloc = jnp.where(valid, rloc, 0)
                for c in range(half):
                    widx_v[pl.ds(c * L, L)] = rloc * sr + (buf_off + c * sc)
                for q in range(L * half // 128):
                    pltpu.async_copy(
                        chunk_s.at[widx_v.at[pl.ds(q * 128, 128)]],
                        wbuf_v.at[pl.ds(q * 128, 128)],
                        sem_g,
                    )
                for q in range(L * half // 128):
                    pltpu.make_async_copy(
                        t3_hbm.at[0, 0, pl.ds(0, 128)],
                        wbuf_v.at[pl.ds(0, 128)],
                        sem_g,
                    ).wait()
                for c in range(half):
                    vals = wbuf_v[pl.ds(c * L, L)]
                    ci = iota * 0 + c
                    plsc.store_scatter(colbuf, [ci, dpos], vals, mask=valid)
                return j + L

            lax.while_loop(batch_cond, batch_body, 0)

        stage(0, 0, CHUNK, sems[0])
        stage(CHUNK, 1, rows_of(1), sems[1])

        @pl.when(sid == 0)
        def _():
            for r in range(128):
                pltpu.async_copy(
                    tail_hbm.at[r, :],
                    chunk_s.at[pl.ds(tail_off + r * 128, 128)],
                    sem_g,
                )
            wait_stage(128, sem_g, count=128)

        def run_epoch(e_idx, k_slot):
            nxt = (k_slot + 2) % NBUF

            @pl.when(e_idx + 2 < n_full)
            def _():
                stage((e_idx + 2) * CHUNK, nxt, CHUNK, sems[nxt])

            @pl.when(e_idx + 2 == n_full)
            def _():
                stage(n_full * CHUNK, nxt, rows2, sems[nxt])

            wait_stage(CHUNK, sems[k_slot])
            plsc.subcore_barrier()
            process(k_slot * cwords, e_idx * CHUNK, CHUNK, 1, CHUNK)

        def super_epoch(t, carry):
            for k_slot in range(NBUF):
                run_epoch(t * NBUF + k_slot, k_slot)
            return carry

        lax.fori_loop(0, n_loop // NBUF, super_epoch, 0)

        for e in range(n_loop, n_all):
            slot = e % NBUF
            rows_e = rows_of(e)
            if e + 2 < n_all:
                stage((e + 2) * CHUNK, (slot + 2) % NBUF, rows_of(e + 2),
                      sems[(slot + 2) % NBUF])
            wait_stage(rows_e, sems[slot])
            plsc.subcore_barrier()
            process(slot * cwords, e * CHUNK, rows_e, 1, rows_e)

        process(tail_off + cid * half, aligned_v, t_rows, 128, 1)

        pltpu.sync_copy(
            colbuf,
            outT_hbm.at[pl.ds(cid * half, half), pl.ds(base, bpt)],
        )

    return k


def kernel(raw_X, table):
    B = raw_X.shape[0]
    V, D = table.shape
    info = plsc.get_sparse_core_info()
    NC, NS = info.num_cores, info.num_subcores
    idx = raw_X[:, 1].astype(jnp.int32)
    t3 = table.T.reshape(D // 8, 8, V)
    aligned_v = (V // 128) * 128
    tail = jnp.pad(
        table[aligned_v:], ((0, 128 - (V - aligned_v)), (0, 128 - D))
    )
    outT = _emb_call(B, V, D, NC, NS)(t3, tail, idx)
    return outT.T


# speedup vs baseline: 3.1616x; 1.0029x over previous
import functools

import jax
import jax.numpy as jnp
from jax import lax
from jax.experimental import pallas as pl
from jax.experimental.pallas import tpu as pltpu
from jax.experimental.pallas import tpu_sc as plsc

L = 16
CHUNK = 11648
PKB = 14
PK = 1 << PKB
NBUF = 4


def _pop(mask):
    return plsc.all_reduce_population_count(mask)[0]


def _emb_call(B, V, D, NC, NS):
    mesh = plsc.VectorSubcoreMesh(core_axis_name="c", subcore_axis_name="s")
    half = D // NC
    bpt = B // NS
    aligned_v = (V // 128) * 128
    n_full = aligned_v // CHUNK
    rows2 = aligned_v - n_full * CHUNK
    n_all = n_full + 1
    n_loop = (n_all - 2) // NBUF * NBUF
    t_rows = V - aligned_v
    cwords = half * CHUNK
    tail_off = NBUF * cwords

    def rows_of(e):
        return CHUNK if e < n_full else rows2

    @functools.partial(
        pl.kernel,
        mesh=mesh,
        out_type=jax.ShapeDtypeStruct((D, B), jnp.float32),
        scratch_types=[
            pltpu.VMEM((bpt,), jnp.int32),
            pltpu.VMEM((bpt + L,), jnp.int32),
            pltpu.VMEM((L * half,), jnp.int32),
            pltpu.VMEM((L * half,), jnp.float32),
            pltpu.VMEM((half, bpt), jnp.float32),
            pltpu.VMEM_SHARED((NBUF * cwords + 128 * 128,), jnp.float32),
            pltpu.SemaphoreType.DMA,
            pltpu.SemaphoreType.DMA,
            pltpu.SemaphoreType.DMA,
            pltpu.SemaphoreType.DMA,
            pltpu.SemaphoreType.DMA,
        ],
        compiler_params=pltpu.CompilerParams(needs_layout_passes=False),
    )
    def k(t3_hbm, tail_hbm, idx_hbm, outT_hbm, idx_v, run_v, widx_v, wbuf_v,
          colbuf, chunk_s, sem0, sem1, sem2, sem3, sem_g):
        sems = [sem0, sem1, sem2, sem3]
        cid = lax.axis_index("c")
        sid = lax.axis_index("s")
        base = sid * bpt
        pltpu.sync_copy(idx_hbm.at[pl.ds(base, bpt)], idx_v)

        iota = lax.iota(jnp.int32, L)

        def stage(lo, slot, rows, sem):
            for j in range(2):
                c_loc = sid * 2 + j
                a = cid * (half // 8) + c_loc // 8
                c2 = c_loc % 8
                pltpu.async_copy(
                    t3_hbm.at[a, c2, pl.ds(lo, rows)],
                    chunk_s.at[pl.ds(slot * cwords + c_loc * rows, rows)],
                    sem,
                )

        def wait_stage(rows, sem, count=2):
            for _ in range(count):
                pltpu.make_async_copy(
                    t3_hbm.at[0, 0, pl.ds(0, rows)],
                    chunk_s.at[pl.ds(0, rows)],
                    sem,
                ).wait()

        def scan(lo, rows):
            def scan_body(g, off):
                v = idx_v[pl.ds(g * L, L)]
                m = (v >= lo) & (v < lo + rows)
                packed = (v - lo) + (iota + g * L) * PK
                plsc.store_compressed(run_v.at[pl.ds(off, L)], packed, mask=m)
                return off + _pop(m)

            return lax.fori_loop(0, bpt // L, scan_body, 0)

        def gather(n, buf_off, sr, sc):
            def batch_cond(j):
                return j < n

            def batch_body(j):
                pv = run_v[pl.ds(j, L)]
                rloc = pv & (PK - 1)
                dpos = pv >> PKB
                valid = (iota + j) < n
                rloc = jnp.where(valid, rloc, 0)
                for c in range(half):
                    widx_v[pl.ds(c * L, L)] = rloc * sr + (buf_off + c * sc)
                for q in range(L * half // 128):
                    pltpu.async_copy(
                        chunk_s.at[widx_v.at[pl.ds(q * 128, 128)]],
                        wbuf_v.at[pl.ds(q * 128, 128)],
                        sem_g,
                    )
                for q in range(L * half // 128):
                    pltpu.make_async_copy(
                        t3_hbm.at[0, 0, pl.ds(0, 128)],
                        wbuf_v.at[pl.ds(0, 128)],
                        sem_g,
                    ).wait()
                for c in range(half):
                    vals = wbuf_v[pl.ds(c * L, L)]
                    ci = iota * 0 + c
                    plsc.store_scatter(colbuf, [ci, dpos], vals, mask=valid)
                return j + L

            lax.while_loop(batch_cond, batch_body, 0)

        stage(0, 0, CHUNK, sems[0])
        stage(CHUNK, 1, rows_of(1), sems[1])

        @pl.when(sid == 0)
        def _():
            for r in range(128):
                pltpu.async_copy(
                    tail_hbm.at[r, :],
                    chunk_s.at[pl.ds(tail_off + r * 128, 128)],
                    sem_g,
                )
            wait_stage(128, sem_g, count=128)

        def run_epoch(e_idx, k_slot):
            nxt = (k_slot + 2) % NBUF

            @pl.when(e_idx + 2 < n_full)
            def _():
                stage((e_idx + 2) * CHUNK, nxt, CHUNK, sems[nxt])

            @pl.when(e_idx + 2 == n_full)
            def _():
                stage(n_full * CHUNK, nxt, rows2, sems[nxt])

            n = scan(e_idx * CHUNK, CHUNK)
            wait_stage(CHUNK, sems[k_slot])
            plsc.subcore_barrier()
            gather(n, k_slot * cwords, 1, CHUNK)

        def super_epoch(t, carry):
            for k_slot in range(NBUF):
                run_epoch(t * NBUF + k_slot, k_slot)
            return carry

        lax.fori_loop(0, n_loop // NBUF, super_epoch, 0)

        for e in range(n_loop, n_all):
            slot = e % NBUF
            rows_e = rows_of(e)
            if e + 2 < n_all:
                stage((e + 2) * CHUNK, (slot + 2) % NBUF, rows_of(e + 2),
                      sems[(slot + 2) % NBUF])
            n = scan(e * CHUNK, rows_e)
            wait_stage(rows_e, sems[slot])
            plsc.subcore_barrier()
            gather(n, slot * cwords, 1, rows_e)

        n = scan(aligned_v, t_rows)
        gather(n, tail_off + cid * half, 128, 1)

        pltpu.sync_copy(
            colbuf,
            outT_hbm.at[pl.ds(cid * half, half), pl.ds(base, bpt)],
        )

    return k


def kernel(raw_X, table):
    B = raw_X.shape[0]
    V, D = table.shape
    info = plsc.get_sparse_core_info()
    NC, NS = info.num_cores, info.num_subcores
    idx = raw_X[:, 1].astype(jnp.int32)
    t3 = table.T.reshape(D // 8, 8, V)
    aligned_v = (V // 128) * 128
    tail = jnp.pad(
        table[aligned_v:], ((0, 128 - (V - aligned_v)), (0, 128 - D))
    )
    outT = _emb_call(B, V, D, NC, NS)(t3, tail, idx)
    return outT.T
